# Initial kernel scaffold; baseline (speedup 1.0000x reference)
import jax
import jax.numpy as jnp
from jax.experimental import pallas as pl


def kernel(x, h, degree, beta, idx_targets, exp_nodes, W_raw, b_raw, W_num, b_num, W1, b1, W2, b2, temperature, epsilon):
    raise NotImplementedError("write your pallas kernel here")


# speedup vs baseline: 2.1936x; 2.1936x over previous
import functools

import jax
import jax.numpy as jnp
from jax import lax
from jax.experimental import pallas as pl
from jax.experimental.pallas import tpu as pltpu
from jax.experimental.pallas import tpu_sc as plsc

MAX_CAND = 128
NEG_INF = float("-inf")


def _sc_gather(x, h, db_tab, exp_pad, idx_targets, per_w, ch):
    n_f = x.shape[1]
    n_d = h.shape[1]
    m_pad = exp_pad.shape[0]
    n_w = 32
    n_ch = per_w // ch
    n_t = idx_targets.shape[0]
    t_per_w = n_t // n_w

    mesh = plsc.VectorSubcoreMesh(core_axis_name="c", subcore_axis_name="s")

    @functools.partial(
        pl.kernel,
        mesh=mesh,
        compiler_params=pltpu.CompilerParams(use_tc_tiling_on_sc=False),
        out_type=[
            jax.ShapeDtypeStruct((m_pad, n_f), jnp.float32),
            jax.ShapeDtypeStruct((m_pad, n_d), jnp.float32),
            jax.ShapeDtypeStruct((m_pad, 16), jnp.float32),
            jax.ShapeDtypeStruct((n_t, n_d), jnp.float32),
        ],
        scratch_types=[
            pltpu.VMEM((ch,), jnp.int32),
            pltpu.VMEM((ch, n_f), jnp.float32),
            pltpu.VMEM((ch, n_d), jnp.float32),
            pltpu.VMEM((ch, 16), jnp.float32),
            pltpu.VMEM((t_per_w,), jnp.int32),
            pltpu.VMEM((t_per_w, n_d), jnp.float32),
            pltpu.SemaphoreType.DMA,
            pltpu.SemaphoreType.DMA,
            pltpu.SemaphoreType.DMA,
        ],
    )
    def gather_kernel(x_hbm, h_hbm, db_hbm, exp_hbm, tgt_hbm,
                      xg_hbm, hg_hbm, dbg_hbm, ht_hbm,
                      idx_v, xbuf, hbuf, dbuf, tidx_v, htbuf,
                      sem0, sem1, sem2):
        wid = lax.axis_index("s") * 2 + lax.axis_index("c")
        base = pl.multiple_of(wid * per_w, 8)

        def body(ci, _):
            off = pl.multiple_of(base + ci * ch, 8)
            pltpu.sync_copy(exp_hbm.at[pl.ds(off, ch)], idx_v)
            cx = pltpu.async_copy(x_hbm.at[idx_v], xbuf, sem0)
            chh = pltpu.async_copy(h_hbm.at[idx_v], hbuf, sem1)
            cd = pltpu.async_copy(db_hbm.at[idx_v], dbuf, sem2)
            cx.wait()
            pltpu.sync_copy(xbuf, xg_hbm.at[pl.ds(off, ch)])
            chh.wait()
            pltpu.sync_copy(hbuf, hg_hbm.at[pl.ds(off, ch)])
            cd.wait()
            pltpu.sync_copy(dbuf, dbg_hbm.at[pl.ds(off, ch)])
            return 0

        lax.fori_loop(0, n_ch, body, 0)

        tb = pl.multiple_of(wid * t_per_w, 8)
        pltpu.sync_copy(tgt_hbm.at[pl.ds(tb, t_per_w)], tidx_v)
        pltpu.async_copy(h_hbm.at[tidx_v], htbuf, sem0).wait()
        pltpu.sync_copy(htbuf, ht_hbm.at[pl.ds(tb, t_per_w)])

    return gather_kernel(x, h, db_tab, exp_pad, idx_targets)


def _tc_score_topk(xg, hg, dbg, ht_rows, exp_cm, W_raw, b_raw, W_num, b_num,
                   W1, b1, W2, b2, m_valid, rows):
    m_pad, n_f = xg.shape
    n_d = hg.shape[1]
    n_t = ht_rows.shape[0]
    n_steps = m_pad // rows

    def _mixed_dot(a_bf16, w_f32):
        return lax.dot_general(
            a_bf16, w_f32, (((1,), (0,)), ((), ())),
            preferred_element_type=jnp.float32)

    def body(xg_ref, hg_ref, dbg_ref, ht_ref, exp_ref,
             wraw_ref, braw_ref, wnum_ref, bnum_ref, w1_ref, b1_ref,
             w2_ref, b2_ref, cand_ref, cidx_ref, s_scr):
        i = pl.program_id(0)

        bf = jnp.bfloat16
        x_v = (_mixed_dot(xg_ref[...].astype(bf), wraw_ref[...])
               + braw_ref[...]).astype(bf)
        h_t = (jnp.sum(ht_ref[...], axis=0, keepdims=True)
               * jnp.float32(1.0 / n_t)).astype(bf)
        emb_num = (_mixed_dot(dbg_ref[...].astype(bf), wnum_ref[...])
                   + bnum_ref[...]).astype(bf)
        emb = jnp.concatenate(
            [x_v, hg_ref[...].astype(bf),
             jnp.broadcast_to(h_t, (rows, n_d)), emb_num], axis=1)
        emb = jnp.maximum(emb, jnp.bfloat16(0.0))
        hid = jnp.maximum(
            _mixed_dot(emb, w1_ref[...]) + b1_ref[...], 0.0).astype(bf)
        s = _mixed_dot(hid, w2_ref[...]) + b2_ref[...]
        lane2 = lax.broadcasted_iota(jnp.int32, (rows, n_steps), 1)
        s_scr[...] = jnp.where(lane2 == i,
                               jnp.broadcast_to(s, (rows, n_steps)),
                               s_scr[...])

        @pl.when(i == n_steps - 1)
        def _():
            lin = (lane2 * rows
                   + lax.broadcasted_iota(jnp.int32, (rows, n_steps), 0))
            s_scr[...] = jnp.where(lin >= m_valid, NEG_INF, s_scr[...])
            out_pos = lax.broadcasted_iota(jnp.int32, (1, MAX_CAND), 1)
            exp_v = exp_ref[...]

            def sel(j, acc):
                sc = s_scr[...]
                m = jnp.max(sc)
                cand_lin = jnp.where(sc == m, lin, jnp.int32(0x7FFFFFFF))
                li = jnp.min(cand_lin)
                hit = lin == li
                s_scr[...] = jnp.where(hit, NEG_INF, sc)
                node = jnp.max(jnp.where(hit, exp_v, jnp.int32(-1)))
                return jnp.where(out_pos == j, node, acc)

            acc = lax.fori_loop(0, MAX_CAND, sel,
                                jnp.zeros((1, MAX_CAND), jnp.int32))
            cidx_ref[...] = acc
            cand_ref[...] = jnp.ones((1, MAX_CAND), jnp.float32)

    out = pl.pallas_call(
        body,
        grid=(n_steps,),
        in_specs=[
            pl.BlockSpec((rows, n_f), lambda i: (i, 0)),
            pl.BlockSpec((rows, n_d), lambda i: (i, 0)),
            pl.BlockSpec((rows, 16), lambda i: (i, 0)),
            pl.BlockSpec((n_t, n_d), lambda i: (0, 0)),
            pl.BlockSpec((rows, n_steps), lambda i: (0, 0)),
            pl.BlockSpec((n_f, n_d), lambda i: (0, 0)),
            pl.BlockSpec((1, n_d), lambda i: (0, 0)),
            pl.BlockSpec((16, n_d), lambda i: (0, 0)),
            pl.BlockSpec((1, n_d), lambda i: (0, 0)),
            pl.BlockSpec((4 * n_d, n_d), lambda i: (0, 0)),
            pl.BlockSpec((1, n_d), lambda i: (0, 0)),
            pl.BlockSpec((n_d, 1), lambda i: (0, 0)),
            pl.BlockSpec((1, 1), lambda i: (0, 0)),
        ],
        out_specs=[
            pl.BlockSpec((1, MAX_CAND), lambda i: (0, 0)),
            pl.BlockSpec((1, MAX_CAND), lambda i: (0, 0)),
        ],
        out_shape=[
            jax.ShapeDtypeStruct((1, MAX_CAND), jnp.float32),
            jax.ShapeDtypeStruct((1, MAX_CAND), jnp.int32),
        ],
        scratch_shapes=[pltpu.VMEM((rows, n_steps), jnp.float32)],
    )(xg, hg, dbg, ht_rows, exp_cm, W_raw, b_raw.reshape(1, -1), W_num,
      b_num.reshape(1, -1), W1, b1.reshape(1, -1), W2, b2.reshape(1, 1))
    return out


def kernel(x, h, degree, beta, idx_targets, exp_nodes,
           W_raw, b_raw, W_num, b_num, W1, b1, W2, b2,
           temperature, epsilon):
    m = exp_nodes.shape[0]
    n_w, ch, rows = 32, 112, 512
    per_w = -(-m // (n_w * ch)) * ch
    m_pad = n_w * per_w
    assert m_pad % rows == 0

    exp_pad = jnp.concatenate(
        [exp_nodes.astype(jnp.int32),
         jnp.zeros((m_pad - m,), jnp.int32)])
    n_nodes = degree.shape[0]
    db_tab = jnp.concatenate(
        [degree[:, None], beta[:, None],
         jnp.zeros((n_nodes, 14), jnp.float32)], axis=1)
    w_num16 = jnp.concatenate(
        [W_num, jnp.zeros((14, W_num.shape[1]), jnp.float32)], axis=0)

    xg, hg, dbg, ht_rows = _sc_gather(
        x, h, db_tab, exp_pad, idx_targets.astype(jnp.int32), per_w, ch)

    n_steps = m_pad // rows
    exp_cm = exp_pad.reshape(n_steps, rows).T

    cand, cidx = _tc_score_topk(
        xg, hg, dbg, ht_rows, exp_cm, W_raw, b_raw, w_num16, b_num,
        W1, b1, W2, b2, m, rows)
    return cand.reshape(MAX_CAND), cidx.reshape(MAX_CAND)


# speedup vs baseline: 2.2703x; 1.0350x over previous
import functools

import jax
import jax.numpy as jnp
from jax import lax
from jax.experimental import pallas as pl
from jax.experimental.pallas import tpu as pltpu
from jax.experimental.pallas import tpu_sc as plsc

MAX_CAND = 128
NEG_INF = float("-inf")
ROWS = 512
N_W = 32
CH = 112


def _sc_gather(x, h, db_tab, exp_pad, idx_targets, per_w):
    n_f = x.shape[1]
    n_d = h.shape[1]
    m_pad = exp_pad.shape[0]
    n_ch = per_w // CH
    n_t = idx_targets.shape[0]
    t_per_w = n_t // N_W

    mesh = plsc.VectorSubcoreMesh(core_axis_name="c", subcore_axis_name="s")

    @functools.partial(
        pl.kernel,
        mesh=mesh,
        compiler_params=pltpu.CompilerParams(use_tc_tiling_on_sc=False),
        out_type=[
            jax.ShapeDtypeStruct((m_pad, n_f), jnp.float32),
            jax.ShapeDtypeStruct((m_pad, n_d), jnp.float32),
            jax.ShapeDtypeStruct((m_pad, 16), jnp.float32),
            jax.ShapeDtypeStruct((n_t, n_d), jnp.float32),
        ],
        scratch_types=[
            pltpu.VMEM((per_w,), jnp.int32),
            pltpu.VMEM((2, CH, n_f), jnp.float32),
            pltpu.VMEM((2, CH, n_d), jnp.float32),
            pltpu.VMEM((2, CH, 16), jnp.float32),
            pltpu.VMEM((t_per_w,), jnp.int32),
            pltpu.VMEM((t_per_w, n_d), jnp.float32),
            pltpu.SemaphoreType.DMA,
            pltpu.SemaphoreType.DMA,
            pltpu.SemaphoreType.DMA,
            pltpu.SemaphoreType.DMA,
            pltpu.SemaphoreType.DMA,
            pltpu.SemaphoreType.DMA,
        ],
    )
    def gather_kernel(x_hbm, h_hbm, db_hbm, exp_hbm, tgt_hbm,
                      xg_hbm, hg_hbm, dbg_hbm, ht_hbm,
                      idx_v, xbuf, hbuf, dbuf, tidx_v, htbuf,
                      sx0, sx1, sh0, sh1, sd0, sd1):
        sems = ((sx0, sh0, sd0), (sx1, sh1, sd1))
        wid = lax.axis_index("s") * 2 + lax.axis_index("c")
        base = pl.multiple_of(wid * per_w, 8)
        pltpu.sync_copy(exp_hbm.at[pl.ds(base, per_w)], idx_v)

        def fire(ci):
            slot = ci & 1
            idx = idx_v.at[pl.ds(ci * CH, CH)]
            sx, sh, sd = sems[slot]
            return (pltpu.async_copy(x_hbm.at[idx], xbuf.at[slot], sx),
                    pltpu.async_copy(h_hbm.at[idx], hbuf.at[slot], sh),
                    pltpu.async_copy(db_hbm.at[idx], dbuf.at[slot], sd))

        def drain_store(ci, handles):
            slot = ci & 1
            off = pl.multiple_of(base + ci * CH, 8)
            cx, chh, cd = handles
            cx.wait()
            pltpu.sync_copy(xbuf.at[slot], xg_hbm.at[pl.ds(off, CH)])
            chh.wait()
            pltpu.sync_copy(hbuf.at[slot], hg_hbm.at[pl.ds(off, CH)])
            cd.wait()
            pltpu.sync_copy(dbuf.at[slot], dbg_hbm.at[pl.ds(off, CH)])

        pend = {0: fire(0)}
        for ci in range(n_ch):
            if ci + 1 < n_ch:
                pend[ci + 1] = fire(ci + 1)
            drain_store(ci, pend.pop(ci))

        tb = pl.multiple_of(wid * t_per_w, 8)
        pltpu.sync_copy(tgt_hbm.at[pl.ds(tb, t_per_w)], tidx_v)
        pltpu.async_copy(h_hbm.at[tidx_v], htbuf, sx0).wait()
        pltpu.sync_copy(htbuf, ht_hbm.at[pl.ds(tb, t_per_w)])

    return gather_kernel(x, h, db_tab, exp_pad, idx_targets)


def _tc_score_topk(xg, hg, dbg, ht_rows, exp_cm, W_raw, b_raw, W_num16,
                   b_num, W1, b1, W2, b2, m_valid):
    m_pad, n_f = xg.shape
    n_d = hg.shape[1]
    n_t = ht_rows.shape[0]
    n_steps = m_pad // ROWS

    def _mixed_dot(a_bf16, w_f32):
        return lax.dot_general(
            a_bf16, w_f32, (((1,), (0,)), ((), ())),
            preferred_element_type=jnp.float32)

    def body(xg_ref, hg_ref, dbg_ref, ht_ref, exp_ref,
             wraw_ref, braw_ref, wnum_ref, bnum_ref, w1_ref, b1_ref,
             w2_ref, b2_ref, cand_ref, cidx_ref, s_scr):
        i = pl.program_id(0)

        bf = jnp.bfloat16
        x_v = (_mixed_dot(xg_ref[...].astype(bf), wraw_ref[...])
               + braw_ref[...]).astype(bf)
        h_t = (jnp.sum(ht_ref[...], axis=0, keepdims=True)
               * jnp.float32(1.0 / n_t)).astype(bf)
        emb_num = (_mixed_dot(dbg_ref[...].astype(bf), wnum_ref[...])
                   + bnum_ref[...]).astype(bf)
        emb = jnp.concatenate(
            [x_v, hg_ref[...].astype(bf),
             jnp.broadcast_to(h_t, (ROWS, n_d)), emb_num], axis=1)
        emb = jnp.maximum(emb, jnp.bfloat16(0.0))
        hid = jnp.maximum(
            _mixed_dot(emb, w1_ref[...]) + b1_ref[...], 0.0).astype(bf)
        s = _mixed_dot(hid, w2_ref[...]) + b2_ref[...]
        lane2 = lax.broadcasted_iota(jnp.int32, (ROWS, n_steps), 1)
        s_scr[...] = jnp.where(lane2 == i,
                               jnp.broadcast_to(s, (ROWS, n_steps)),
                               s_scr[...])

        @pl.when(i == n_steps - 1)
        def _():
            lin = (lane2 * ROWS
                   + lax.broadcasted_iota(jnp.int32, (ROWS, n_steps), 0))
            s_scr[...] = jnp.where(lin >= m_valid, NEG_INF, s_scr[...])
            out_pos = lax.broadcasted_iota(jnp.int32, (1, MAX_CAND), 1)
            exp_v = exp_ref[...]

            def sel(j, acc):
                sc = s_scr[...]
                m = jnp.max(sc)
                cand_lin = jnp.where(sc == m, lin, jnp.int32(0x7FFFFFFF))
                li = jnp.min(cand_lin)
                hit = lin == li
                s_scr[...] = jnp.where(hit, NEG_INF, sc)
                node = jnp.max(jnp.where(hit, exp_v, jnp.int32(-1)))
                return jnp.where(out_pos == j, node, acc)

            acc = lax.fori_loop(0, MAX_CAND, sel,
                                jnp.zeros((1, MAX_CAND), jnp.int32))
            cidx_ref[...] = acc
            cand_ref[...] = jnp.ones((1, MAX_CAND), jnp.float32)

    out = pl.pallas_call(
        body,
        grid=(n_steps,),
        in_specs=[
            pl.BlockSpec((ROWS, n_f), lambda i: (i, 0)),
            pl.BlockSpec((ROWS, n_d), lambda i: (i, 0)),
            pl.BlockSpec((ROWS, 16), lambda i: (i, 0)),
            pl.BlockSpec((n_t, n_d), lambda i: (0, 0)),
            pl.BlockSpec((ROWS, n_steps), lambda i: (0, 0)),
            pl.BlockSpec((n_f, n_d), lambda i: (0, 0)),
            pl.BlockSpec((1, n_d), lambda i: (0, 0)),
            pl.BlockSpec((16, n_d), lambda i: (0, 0)),
            pl.BlockSpec((1, n_d), lambda i: (0, 0)),
            pl.BlockSpec((4 * n_d, n_d), lambda i: (0, 0)),
            pl.BlockSpec((1, n_d), lambda i: (0, 0)),
            pl.BlockSpec((n_d, 1), lambda i: (0, 0)),
            pl.BlockSpec((1, 1), lambda i: (0, 0)),
        ],
        out_specs=[
            pl.BlockSpec((1, MAX_CAND), lambda i: (0, 0)),
            pl.BlockSpec((1, MAX_CAND), lambda i: (0, 0)),
        ],
        out_shape=[
            jax.ShapeDtypeStruct((1, MAX_CAND), jnp.float32),
            jax.ShapeDtypeStruct((1, MAX_CAND), jnp.int32),
        ],
        scratch_shapes=[pltpu.VMEM((ROWS, n_steps), jnp.float32)],
    )(xg, hg, dbg, ht_rows, exp_cm, W_raw, b_raw.reshape(1, -1), W_num16,
      b_num.reshape(1, -1), W1, b1.reshape(1, -1), W2, b2.reshape(1, 1))
    return out


def kernel(x, h, degree, beta, idx_targets, exp_nodes,
           W_raw, b_raw, W_num, b_num, W1, b1, W2, b2,
           temperature, epsilon):
    m = exp_nodes.shape[0]
    per_w = -(-m // (N_W * CH)) * CH
    m_pad = N_W * per_w
    assert m_pad % ROWS == 0

    exp_pad = jnp.concatenate(
        [exp_nodes.astype(jnp.int32),
         jnp.zeros((m_pad - m,), jnp.int32)])
    n_nodes = degree.shape[0]
    db_tab = jnp.concatenate(
        [degree[:, None], beta[:, None],
         jnp.zeros((n_nodes, 14), jnp.float32)], axis=1)
    w_num16 = jnp.concatenate(
        [W_num, jnp.zeros((14, W_num.shape[1]), jnp.float32)], axis=0)

    xg, hg, dbg, ht_rows = _sc_gather(
        x, h, db_tab, exp_pad, idx_targets.astype(jnp.int32), per_w)

    n_steps = m_pad // ROWS
    exp_cm = exp_pad.reshape(n_steps, ROWS).T

    cand, cidx = _tc_score_topk(
        xg, hg, dbg, ht_rows, exp_cm, W_raw, b_raw, w_num16, b_num,
        W1, b1, W2, b2, m)
    return cand.reshape(MAX_CAND), cidx.reshape(MAX_CAND)


# speedup vs baseline: 2.3697x; 1.0438x over previous
import functools

import jax
import jax.numpy as jnp
from jax import lax
from jax.experimental import pallas as pl
from jax.experimental.pallas import tpu as pltpu
from jax.experimental.pallas import tpu_sc as plsc

MAX_CAND = 128
NEG_INF = float("-inf")
ROWS = 1024
N_W = 32
CH = 112


def _sc_gather(x, h, db_tab, exp_pad, idx_targets, per_w):
    n_f = x.shape[1]
    n_d = h.shape[1]
    m_pad = exp_pad.shape[0]
    n_ch = per_w // CH
    n_t = idx_targets.shape[0]
    t_per_w = n_t // N_W

    mesh = plsc.VectorSubcoreMesh(core_axis_name="c", subcore_axis_name="s")

    @functools.partial(
        pl.kernel,
        mesh=mesh,
        compiler_params=pltpu.CompilerParams(use_tc_tiling_on_sc=False),
        out_type=[
            jax.ShapeDtypeStruct((m_pad, n_f), jnp.float32),
            jax.ShapeDtypeStruct((m_pad, n_d), jnp.float32),
            jax.ShapeDtypeStruct((m_pad, 16), jnp.float32),
            jax.ShapeDtypeStruct((n_t, n_d), jnp.float32),
        ],
        scratch_types=[
            pltpu.VMEM((per_w,), jnp.int32),
            pltpu.VMEM((2, CH, n_f), jnp.float32),
            pltpu.VMEM((2, CH, n_d), jnp.float32),
            pltpu.VMEM((2, CH, 16), jnp.float32),
            pltpu.VMEM((t_per_w,), jnp.int32),
            pltpu.VMEM((t_per_w, n_d), jnp.float32),
            pltpu.SemaphoreType.DMA,
            pltpu.SemaphoreType.DMA,
            pltpu.SemaphoreType.DMA,
            pltpu.SemaphoreType.DMA,
            pltpu.SemaphoreType.DMA,
            pltpu.SemaphoreType.DMA,
        ],
    )
    def gather_kernel(x_hbm, h_hbm, db_hbm, exp_hbm, tgt_hbm,
                      xg_hbm, hg_hbm, dbg_hbm, ht_hbm,
                      idx_v, xbuf, hbuf, dbuf, tidx_v, htbuf,
                      sx0, sx1, sh0, sh1, sd0, sd1):
        sems = ((sx0, sh0, sd0), (sx1, sh1, sd1))
        wid = lax.axis_index("s") * 2 + lax.axis_index("c")
        base = pl.multiple_of(wid * per_w, 8)
        pltpu.sync_copy(exp_hbm.at[pl.ds(base, per_w)], idx_v)

        def fire(ci):
            slot = ci & 1
            idx = idx_v.at[pl.ds(ci * CH, CH)]
            sx, sh, sd = sems[slot]
            return (pltpu.async_copy(x_hbm.at[idx], xbuf.at[slot], sx),
                    pltpu.async_copy(h_hbm.at[idx], hbuf.at[slot], sh),
                    pltpu.async_copy(db_hbm.at[idx], dbuf.at[slot], sd))

        def drain_store(ci, handles):
            slot = ci & 1
            off = pl.multiple_of(base + ci * CH, 8)
            cx, chh, cd = handles
            cx.wait()
            pltpu.sync_copy(xbuf.at[slot], xg_hbm.at[pl.ds(off, CH)])
            chh.wait()
            pltpu.sync_copy(hbuf.at[slot], hg_hbm.at[pl.ds(off, CH)])
            cd.wait()
            pltpu.sync_copy(dbuf.at[slot], dbg_hbm.at[pl.ds(off, CH)])

        pend = {0: fire(0)}
        for ci in range(n_ch):
            if ci + 1 < n_ch:
                pend[ci + 1] = fire(ci + 1)
            drain_store(ci, pend.pop(ci))

        tb = pl.multiple_of(wid * t_per_w, 8)
        pltpu.sync_copy(tgt_hbm.at[pl.ds(tb, t_per_w)], tidx_v)
        pltpu.async_copy(h_hbm.at[tidx_v], htbuf, sx0).wait()
        pltpu.sync_copy(htbuf, ht_hbm.at[pl.ds(tb, t_per_w)])

    return gather_kernel(x, h, db_tab, exp_pad, idx_targets)


def _tc_score_topk(xg, hg, dbg, ht_rows, exp_cm, W_raw, b_raw, W_num16,
                   b_num, W1, b1, W2, b2, m_valid):
    m_pad, n_f = xg.shape
    n_d = hg.shape[1]
    n_t = ht_rows.shape[0]
    n_steps = m_pad // ROWS

    def _mixed_dot(a_bf16, w_f32):
        return lax.dot_general(
            a_bf16, w_f32, (((1,), (0,)), ((), ())),
            preferred_element_type=jnp.float32)

    def body(xg_ref, hg_ref, dbg_ref, ht_ref, exp_ref,
             wraw_ref, braw_ref, wnum_ref, bnum_ref, w1_ref, b1_ref,
             w2_ref, b2_ref, cand_ref, cidx_ref, s_scr):
        i = pl.program_id(0)

        bf = jnp.bfloat16
        x_v = (_mixed_dot(xg_ref[...].astype(bf), wraw_ref[...])
               + braw_ref[...]).astype(bf)
        h_t = (jnp.sum(ht_ref[...], axis=0, keepdims=True)
               * jnp.float32(1.0 / n_t)).astype(bf)
        emb_num = (_mixed_dot(dbg_ref[...].astype(bf), wnum_ref[...])
                   + bnum_ref[...]).astype(bf)
        emb = jnp.concatenate(
            [x_v, hg_ref[...].astype(bf),
             jnp.broadcast_to(h_t, (ROWS, n_d)), emb_num], axis=1)
        emb = jnp.maximum(emb, jnp.bfloat16(0.0))
        hid = jnp.maximum(
            _mixed_dot(emb, w1_ref[...]) + b1_ref[...], 0.0).astype(bf)
        s = _mixed_dot(hid, w2_ref[...]) + b2_ref[...]
        lane2 = lax.broadcasted_iota(jnp.int32, (ROWS, n_steps), 1)
        s_scr[...] = jnp.where(lane2 == i,
                               jnp.broadcast_to(s, (ROWS, n_steps)),
                               s_scr[...])

        @pl.when(i == n_steps - 1)
        def _():
            lin = (lane2 * ROWS
                   + lax.broadcasted_iota(jnp.int32, (ROWS, n_steps), 0))
            s_scr[...] = jnp.where(lin >= m_valid, NEG_INF, s_scr[...])
            out_pos = lax.broadcasted_iota(jnp.int32, (1, MAX_CAND), 1)
            exp_v = exp_ref[...]

            def sel(j, acc):
                sc = s_scr[...]
                m = jnp.max(sc)
                cand_lin = jnp.where(sc == m, lin, jnp.int32(0x7FFFFFFF))
                li = jnp.min(cand_lin)
                hit = lin == li
                s_scr[...] = jnp.where(hit, NEG_INF, sc)
                node = jnp.max(jnp.where(hit, exp_v, jnp.int32(-1)))
                return jnp.where(out_pos == j, node, acc)

            acc = lax.fori_loop(0, MAX_CAND, sel,
                                jnp.zeros((1, MAX_CAND), jnp.int32))
            cidx_ref[...] = acc
            cand_ref[...] = jnp.ones((1, MAX_CAND), jnp.float32)

    out = pl.pallas_call(
        body,
        grid=(n_steps,),
        in_specs=[
            pl.BlockSpec((ROWS, n_f), lambda i: (i, 0)),
            pl.BlockSpec((ROWS, n_d), lambda i: (i, 0)),
            pl.BlockSpec((ROWS, 16), lambda i: (i, 0)),
            pl.BlockSpec((n_t, n_d), lambda i: (0, 0)),
            pl.BlockSpec((ROWS, n_steps), lambda i: (0, 0)),
            pl.BlockSpec((n_f, n_d), lambda i: (0, 0)),
            pl.BlockSpec((1, n_d), lambda i: (0, 0)),
            pl.BlockSpec((16, n_d), lambda i: (0, 0)),
            pl.BlockSpec((1, n_d), lambda i: (0, 0)),
            pl.BlockSpec((4 * n_d, n_d), lambda i: (0, 0)),
            pl.BlockSpec((1, n_d), lambda i: (0, 0)),
            pl.BlockSpec((n_d, 1), lambda i: (0, 0)),
            pl.BlockSpec((1, 1), lambda i: (0, 0)),
        ],
        out_specs=[
            pl.BlockSpec((1, MAX_CAND), lambda i: (0, 0)),
            pl.BlockSpec((1, MAX_CAND), lambda i: (0, 0)),
        ],
        out_shape=[
            jax.ShapeDtypeStruct((1, MAX_CAND), jnp.float32),
            jax.ShapeDtypeStruct((1, MAX_CAND), jnp.int32),
        ],
        scratch_shapes=[pltpu.VMEM((ROWS, n_steps), jnp.float32)],
    )(xg, hg, dbg, ht_rows, exp_cm, W_raw, b_raw.reshape(1, -1), W_num16,
      b_num.reshape(1, -1), W1, b1.reshape(1, -1), W2, b2.reshape(1, 1))
    return out


def kernel(x, h, degree, beta, idx_targets, exp_nodes,
           W_raw, b_raw, W_num, b_num, W1, b1, W2, b2,
           temperature, epsilon):
    m = exp_nodes.shape[0]
    per_w = -(-m // (N_W * CH)) * CH
    m_pad = N_W * per_w
    assert m_pad % ROWS == 0

    exp_pad = jnp.concatenate(
        [exp_nodes.astype(jnp.int32),
         jnp.zeros((m_pad - m,), jnp.int32)])
    n_nodes = degree.shape[0]
    db_tab = jnp.concatenate(
        [degree[:, None], beta[:, None],
         jnp.zeros((n_nodes, 14), jnp.float32)], axis=1)
    w_num16 = jnp.concatenate(
        [W_num, jnp.zeros((14, W_num.shape[1]), jnp.float32)], axis=0)

    xg, hg, dbg, ht_rows = _sc_gather(
        x, h, db_tab, exp_pad, idx_targets.astype(jnp.int32), per_w)

    n_steps = m_pad // ROWS
    exp_cm = exp_pad.reshape(n_steps, ROWS).T

    cand, cidx = _tc_score_topk(
        xg, hg, dbg, ht_rows, exp_cm, W_raw, b_raw, w_num16, b_num,
        W1, b1, W2, b2, m)
    return cand.reshape(MAX_CAND), cidx.reshape(MAX_CAND)


# speedup vs baseline: 2.4165x; 1.0198x over previous
import functools

import jax
import jax.numpy as jnp
from jax import lax
from jax.experimental import pallas as pl
from jax.experimental.pallas import tpu as pltpu
from jax.experimental.pallas import tpu_sc as plsc

MAX_CAND = 128
NEG_INF = float("-inf")
ROWS = 1024
N_W = 32
CH = 112


def _sc_gather(x, h, db_tab, exp_pad, idx_targets, per_w):
    n_f = x.shape[1]
    n_d = h.shape[1]
    m_pad = exp_pad.shape[0]
    n_ch = per_w // CH
    n_t = idx_targets.shape[0]
    t_per_w = n_t // N_W

    mesh = plsc.VectorSubcoreMesh(core_axis_name="c", subcore_axis_name="s")

    @functools.partial(
        pl.kernel,
        mesh=mesh,
        compiler_params=pltpu.CompilerParams(use_tc_tiling_on_sc=False),
        out_type=[
            jax.ShapeDtypeStruct((m_pad, n_f), jnp.float32),
            jax.ShapeDtypeStruct((m_pad, n_d), jnp.float32),
            jax.ShapeDtypeStruct((m_pad, 16), jnp.float32),
            jax.ShapeDtypeStruct((n_t, n_d), jnp.float32),
        ],
        scratch_types=[
            pltpu.VMEM((per_w,), jnp.int32),
            pltpu.VMEM((2, CH, n_f), jnp.float32),
            pltpu.VMEM((2, CH, n_d), jnp.float32),
            pltpu.VMEM((2, CH, 16), jnp.float32),
            pltpu.VMEM((t_per_w,), jnp.int32),
            pltpu.VMEM((t_per_w, n_d), jnp.float32),
            pltpu.SemaphoreType.DMA,
            pltpu.SemaphoreType.DMA,
            pltpu.SemaphoreType.DMA,
            pltpu.SemaphoreType.DMA,
            pltpu.SemaphoreType.DMA,
            pltpu.SemaphoreType.DMA,
        ],
    )
    def gather_kernel(x_hbm, h_hbm, db_hbm, exp_hbm, tgt_hbm,
                      xg_hbm, hg_hbm, dbg_hbm, ht_hbm,
                      idx_v, xbuf, hbuf, dbuf, tidx_v, htbuf,
                      sx0, sx1, sh0, sh1, sd0, sd1):
        sems = ((sx0, sh0, sd0), (sx1, sh1, sd1))
        wid = lax.axis_index("s") * 2 + lax.axis_index("c")
        base = pl.multiple_of(wid * per_w, 8)
        pltpu.sync_copy(exp_hbm.at[pl.ds(base, per_w)], idx_v)

        def fire(ci):
            slot = ci & 1
            idx = idx_v.at[pl.ds(ci * CH, CH)]
            sx, sh, sd = sems[slot]
            return (pltpu.async_copy(x_hbm.at[idx], xbuf.at[slot], sx),
                    pltpu.async_copy(h_hbm.at[idx], hbuf.at[slot], sh),
                    pltpu.async_copy(db_hbm.at[idx], dbuf.at[slot], sd))

        def drain_store(ci, handles):
            slot = ci & 1
            off = pl.multiple_of(base + ci * CH, 8)
            cx, chh, cd = handles
            cx.wait()
            pltpu.sync_copy(xbuf.at[slot], xg_hbm.at[pl.ds(off, CH)])
            chh.wait()
            pltpu.sync_copy(hbuf.at[slot], hg_hbm.at[pl.ds(off, CH)])
            cd.wait()
            pltpu.sync_copy(dbuf.at[slot], dbg_hbm.at[pl.ds(off, CH)])

        pend = {0: fire(0)}
        for ci in range(n_ch):
            if ci + 1 < n_ch:
                pend[ci + 1] = fire(ci + 1)
            drain_store(ci, pend.pop(ci))

        tb = pl.multiple_of(wid * t_per_w, 8)
        pltpu.sync_copy(tgt_hbm.at[pl.ds(tb, t_per_w)], tidx_v)
        pltpu.async_copy(h_hbm.at[tidx_v], htbuf, sx0).wait()
        pltpu.sync_copy(htbuf, ht_hbm.at[pl.ds(tb, t_per_w)])

    return gather_kernel(x, h, db_tab, exp_pad, idx_targets)


def _tc_score_topk(xg, hg, dbg, ht_rows, exp_cm, W_raw, b_raw, W_num16,
                   b_num, W1, b1, W2, b2, m_valid):
    m_pad, n_f = xg.shape
    n_d = hg.shape[1]
    n_t = ht_rows.shape[0]
    n_steps = m_pad // ROWS

    def _mixed_dot(a_bf16, w_f32):
        return lax.dot_general(
            a_bf16, w_f32, (((1,), (0,)), ((), ())),
            preferred_element_type=jnp.float32)

    def body(xg_ref, hg_ref, dbg_ref, ht_ref, exp_ref,
             wraw_ref, braw_ref, wnum_ref, bnum_ref, w1_ref, b1_ref,
             w2_ref, b2_ref, cand_ref, cidx_ref, s_scr):
        i = pl.program_id(0)

        bf = jnp.bfloat16
        x_v = (_mixed_dot(xg_ref[...].astype(bf), wraw_ref[...])
               + braw_ref[...]).astype(bf)
        h_t = (jnp.sum(ht_ref[...], axis=0, keepdims=True)
               * jnp.float32(1.0 / n_t)).astype(bf)
        emb_num = (_mixed_dot(dbg_ref[...].astype(bf), wnum_ref[...])
                   + bnum_ref[...]).astype(bf)
        emb = jnp.concatenate(
            [x_v, hg_ref[...].astype(bf),
             jnp.broadcast_to(h_t, (ROWS, n_d)), emb_num], axis=1)
        emb = jnp.maximum(emb, jnp.bfloat16(0.0))
        hid = jnp.maximum(
            _mixed_dot(emb, w1_ref[...]) + b1_ref[...], 0.0).astype(bf)
        s = _mixed_dot(hid, w2_ref[...]) + b2_ref[...]
        lane2 = lax.broadcasted_iota(jnp.int32, (ROWS, n_steps), 1)
        s_scr[...] = jnp.where(lane2 == i,
                               jnp.broadcast_to(s, (ROWS, n_steps)),
                               s_scr[...])

        @pl.when(i == n_steps - 1)
        def _():
            lin = (lane2 * ROWS
                   + lax.broadcasted_iota(jnp.int32, (ROWS, n_steps), 0))
            s_scr[...] = jnp.where(lin >= m_valid, NEG_INF, s_scr[...])
            out_pos = lax.broadcasted_iota(jnp.int32, (1, MAX_CAND), 1)
            big = jnp.int32(0x7FFFFFFF)

            n_grp = ROWS // 8
            s3 = s_scr[...].reshape(n_grp, 8, n_steps)
            g0 = s3[:, 0, :]
            for d in range(1, 8):
                g0 = jnp.maximum(g0, s3[:, d, :])
            glin = (lax.broadcasted_iota(jnp.int32, (n_grp, n_steps), 1)
                    * n_grp
                    + lax.broadcasted_iota(jnp.int32, (n_grp, n_steps), 0))
            row8 = lax.broadcasted_iota(jnp.int32, (8, n_steps), 0)
            col8 = lax.broadcasted_iota(jnp.int32, (8, n_steps), 1)

            def sel(j, carry):
                grp, acc = carry
                m = jnp.max(grp)
                gmin = jnp.min(jnp.where(grp == m, glin, big))
                g = gmin % n_grp
                slab = s_scr[pl.ds(g * 8, 8), :]
                slab_lin = col8 * ROWS + (g * 8 + row8)
                li = jnp.min(jnp.where(slab == m, slab_lin, big))
                slab2 = jnp.where(slab_lin == li, NEG_INF, slab)
                s_scr[pl.ds(g * 8, 8), :] = slab2
                eslab = exp_ref[pl.ds(g * 8, 8), :]
                node = jnp.max(jnp.where(slab_lin == li, eslab,
                                         jnp.int32(-1)))
                newg = jnp.max(slab2, axis=0, keepdims=True)
                grp = jnp.where(glin % n_grp == g,
                                jnp.broadcast_to(newg, (n_grp, n_steps)),
                                grp)
                acc = jnp.where(out_pos == j, node, acc)
                return grp, acc

            _, acc = lax.fori_loop(
                0, MAX_CAND, sel,
                (g0, jnp.zeros((1, MAX_CAND), jnp.int32)))
            cidx_ref[...] = acc
            cand_ref[...] = jnp.ones((1, MAX_CAND), jnp.float32)

    out = pl.pallas_call(
        body,
        grid=(n_steps,),
        in_specs=[
            pl.BlockSpec((ROWS, n_f), lambda i: (i, 0)),
            pl.BlockSpec((ROWS, n_d), lambda i: (i, 0)),
            pl.BlockSpec((ROWS, 16), lambda i: (i, 0)),
            pl.BlockSpec((n_t, n_d), lambda i: (0, 0)),
            pl.BlockSpec((ROWS, n_steps), lambda i: (0, 0)),
            pl.BlockSpec((n_f, n_d), lambda i: (0, 0)),
            pl.BlockSpec((1, n_d), lambda i: (0, 0)),
            pl.BlockSpec((16, n_d), lambda i: (0, 0)),
            pl.BlockSpec((1, n_d), lambda i: (0, 0)),
            pl.BlockSpec((4 * n_d, n_d), lambda i: (0, 0)),
            pl.BlockSpec((1, n_d), lambda i: (0, 0)),
            pl.BlockSpec((n_d, 1), lambda i: (0, 0)),
            pl.BlockSpec((1, 1), lambda i: (0, 0)),
        ],
        out_specs=[
            pl.BlockSpec((1, MAX_CAND), lambda i: (0, 0)),
            pl.BlockSpec((1, MAX_CAND), lambda i: (0, 0)),
        ],
        out_shape=[
            jax.ShapeDtypeStruct((1, MAX_CAND), jnp.float32),
            jax.ShapeDtypeStruct((1, MAX_CAND), jnp.int32),
        ],
        scratch_shapes=[pltpu.VMEM((ROWS, n_steps), jnp.float32)],
    )(xg, hg, dbg, ht_rows, exp_cm, W_raw, b_raw.reshape(1, -1), W_num16,
      b_num.reshape(1, -1), W1, b1.reshape(1, -1), W2, b2.reshape(1, 1))
    return out


def kernel(x, h, degree, beta, idx_targets, exp_nodes,
           W_raw, b_raw, W_num, b_num, W1, b1, W2, b2,
           temperature, epsilon):
    m = exp_nodes.shape[0]
    per_w = -(-m // (N_W * CH)) * CH
    m_pad = N_W * per_w
    assert m_pad % ROWS == 0

    exp_pad = jnp.concatenate(
        [exp_nodes.astype(jnp.int32),
         jnp.zeros((m_pad - m,), jnp.int32)])
    n_nodes = degree.shape[0]
    db_tab = jnp.concatenate(
        [degree[:, None], beta[:, None],
         jnp.zeros((n_nodes, 14), jnp.float32)], axis=1)
    w_num16 = jnp.concatenate(
        [W_num, jnp.zeros((14, W_num.shape[1]), jnp.float32)], axis=0)

    xg, hg, dbg, ht_rows = _sc_gather(
        x, h, db_tab, exp_pad, idx_targets.astype(jnp.int32), per_w)

    n_steps = m_pad // ROWS
    exp_cm = exp_pad.reshape(n_steps, ROWS).T

    cand, cidx = _tc_score_topk(
        xg, hg, dbg, ht_rows, exp_cm, W_raw, b_raw, w_num16, b_num,
        W1, b1, W2, b2, m)
    return cand.reshape(MAX_CAND), cidx.reshape(MAX_CAND)
